# baseline (speedup 1.0000x reference)
import functools

import jax
import jax.numpy as jnp
from jax import lax
from jax.experimental import pallas as pl
from jax.experimental.pallas import tpu as pltpu
from jax.experimental.pallas import tpu_sc as plsc

_NCATS = 100000
_BATCH = 1024
_HIST = 200
_LANES = 16
_SPLIT = 49920
_C0 = _SPLIT
_C1 = _NCATS - _SPLIT
_BUF = _C1

_info = plsc.get_sparse_core_info()
_NC = _info.num_cores
_NW = _NC * _info.num_subcores
_ROWS_PER_W = _BATCH // _NW

_WINDOWS = [j * _LANES for j in range(_HIST // _LANES)]
if _HIST % _LANES:
    _WINDOWS.append(_HIST - _LANES)

_mesh = plsc.VectorSubcoreMesh(core_axis_name="c", subcore_axis_name="s")


@functools.partial(
    pl.kernel,
    mesh=_mesh,
    out_type=jax.ShapeDtypeStruct((_BATCH, _NCATS), jnp.float32),
    scratch_types=[
        pltpu.VMEM((_BUF,), jnp.float32),
        pltpu.VMEM((_BUF,), jnp.float32),
        pltpu.VMEM((_ROWS_PER_W, _HIST), jnp.int32),
        pltpu.SemaphoreType.DMA,
        pltpu.SemaphoreType.DMA,
        pltpu.SemaphoreType.DMA,
    ],
    compiler_params=pltpu.CompilerParams(needs_layout_passes=False),
)
def _encode(x_hbm, out_hbm, buf_a, buf_b, idx_v, sem_a, sem_b, sem_i):
    wid = lax.axis_index("s") * _NC + lax.axis_index("c")
    row0 = wid * _ROWS_PER_W
    zeros16 = jnp.zeros((_LANES,), jnp.float32)
    ones16 = jnp.ones((_LANES,), jnp.float32)

    idx_copy = pltpu.make_async_copy(
        x_hbm.at[pl.ds(row0, _ROWS_PER_W)], idx_v, sem_i
    )
    idx_copy.start()

    def zero_body(i, carry):
        base = i * (5 * _LANES)
        for j in range(5):
            buf_a[pl.ds(base + j * _LANES, _LANES)] = zeros16
            buf_b[pl.ds(base + j * _LANES, _LANES)] = zeros16
        return carry

    lax.fori_loop(0, _BUF // (5 * _LANES), zero_body, 0)
    idx_copy.wait()

    def scatter_row(r, buf, half, val16):
        for off in _WINDOWS:
            v = idx_v[r, pl.ds(off, _LANES)]
            if half == 0:
                m = v < _SPLIT
                local = v
            else:
                m = v >= _SPLIT
                local = v - _SPLIT
            plsc.store_scatter(
                buf, [jnp.where(m, local, 0)], val16, mask=m
            )

    def row_body(r, carry):
        row = row0 + r

        @pl.when(r > 0)
        def _():
            pltpu.make_async_copy(
                buf_a.at[pl.ds(0, _C0)], out_hbm.at[row, pl.ds(0, _C0)], sem_a
            ).wait()
            scatter_row(r - 1, buf_a, 0, zeros16)

        scatter_row(r, buf_a, 0, ones16)
        pltpu.make_async_copy(
            buf_a.at[pl.ds(0, _C0)], out_hbm.at[row, pl.ds(0, _C0)], sem_a
        ).start()

        @pl.when(r > 0)
        def _():
            pltpu.make_async_copy(
                buf_b, out_hbm.at[row, pl.ds(_SPLIT, _C1)], sem_b
            ).wait()
            scatter_row(r - 1, buf_b, 1, zeros16)

        scatter_row(r, buf_b, 1, ones16)
        pltpu.make_async_copy(
            buf_b, out_hbm.at[row, pl.ds(_SPLIT, _C1)], sem_b
        ).start()
        return carry

    lax.fori_loop(0, _ROWS_PER_W, row_body, 0)

    last = row0 + _ROWS_PER_W - 1
    pltpu.make_async_copy(
        buf_a.at[pl.ds(0, _C0)], out_hbm.at[last, pl.ds(0, _C0)], sem_a
    ).wait()
    pltpu.make_async_copy(
        buf_b, out_hbm.at[last, pl.ds(_SPLIT, _C1)], sem_b
    ).wait()


def kernel(x):
    return _encode(x)
